# Initial kernel scaffold; baseline (speedup 1.0000x reference)
import jax
import jax.numpy as jnp
from jax.experimental import pallas as pl


def kernel(x, edge_index, W1, b1, W2, b2):
    raise NotImplementedError("write your pallas kernel here")


# speedup vs baseline: 15.5346x; 15.5346x over previous
import jax
import jax.numpy as jnp
from jax import lax
from jax.experimental import pallas as pl
from jax.experimental.pallas import tpu as pltpu
from jax.experimental.pallas import tpu_sc as plsc

N = 10000
NC = 2
NS = 16
NW = NC * NS
CW = 128
NPAD = 10240
RPT = NPAD // NS
BM = 256


def _mesh():
    return plsc.VectorSubcoreMesh(
        core_axis_name="c", subcore_axis_name="s", num_cores=NC, num_subcores=NS
    )


def _make_deg_kernel(chunks):
    def body(dst_hbm, zeros_hbm, ones_hbm, out_hbm, idx_v, ones_v, deg_sh):
        c = lax.axis_index("c")
        s = lax.axis_index("s")
        wid = c * NS + s
        pltpu.sync_copy(dst_hbm.at[wid], idx_v)
        pltpu.sync_copy(ones_hbm, ones_v)
        for k in range(RPT // CW):
            pltpu.sync_copy(
                zeros_hbm.at[k], deg_sh.at[pl.ds(s * RPT + k * CW, CW)]
            )
        plsc.subcore_barrier()
        for j in range(chunks):
            pltpu.sync_copy(ones_v, deg_sh.at[idx_v.at[j]], add=True)
        plsc.subcore_barrier()
        pltpu.sync_copy(
            deg_sh.at[pl.ds(s * RPT, RPT)], out_hbm.at[c, pl.ds(s * RPT, RPT)]
        )

    return pl.kernel(
        body,
        out_type=jax.ShapeDtypeStruct((NC, NPAD), jnp.float32),
        mesh=_mesh(),
        scratch_types=[
            pltpu.VMEM((chunks, CW), jnp.int32),
            pltpu.VMEM((CW,), jnp.float32),
            pltpu.VMEM_SHARED((NPAD,), jnp.float32),
        ],
    )


def _make_scatter_kernel(chunks, dh):
    d2 = dh // 2
    assert chunks % 2 == 0

    def body(y_hbm, src_e_hbm, src_o_hbm, dst_hbm, zeros_hbm, out_hbm,
             srcv, dstv, rows0, rows1, zbuf, sem0, sem1, acc_sh):
        c = lax.axis_index("c")
        s = lax.axis_index("s")
        @pl.when(c == 0)
        def _():
            pltpu.sync_copy(src_e_hbm.at[s], srcv)

        @pl.when(c == 1)
        def _():
            pltpu.sync_copy(src_o_hbm.at[s], srcv)

        pltpu.sync_copy(dst_hbm.at[s], dstv)
        pltpu.sync_copy(zeros_hbm, zbuf)
        for k in range(RPT // CW):
            pltpu.sync_copy(zbuf, acc_sh.at[pl.ds(s * RPT + k * CW, CW)])
        plsc.subcore_barrier()

        pltpu.async_copy(y_hbm.at[srcv.at[0]], rows0, sem0)

        def step(t, carry):
            pltpu.async_copy(y_hbm.at[srcv.at[2 * t + 1]], rows1, sem1)
            pltpu.make_async_copy(y_hbm.at[srcv.at[0]], rows0, sem0).wait()
            pltpu.sync_copy(rows0, acc_sh.at[dstv.at[2 * t]], add=True)

            @pl.when(t + 1 < chunks // 2)
            def _():
                pltpu.async_copy(y_hbm.at[srcv.at[2 * t + 2]], rows0, sem0)

            pltpu.make_async_copy(y_hbm.at[srcv.at[0]], rows1, sem1).wait()
            pltpu.sync_copy(rows1, acc_sh.at[dstv.at[2 * t + 1]], add=True)
            return carry

        lax.fori_loop(0, chunks // 2, step, 0)

        plsc.subcore_barrier()
        pltpu.sync_copy(
            acc_sh.at[pl.ds(s * RPT, RPT)], out_hbm.at[c, pl.ds(s * RPT, RPT)]
        )

    return pl.kernel(
        body,
        out_type=jax.ShapeDtypeStruct((NC, NPAD, d2), jnp.float32),
        mesh=_mesh(),
        compiler_params=pltpu.CompilerParams(use_tc_tiling_on_sc=False),
        scratch_types=[
            pltpu.VMEM((chunks, CW), jnp.int32),
            pltpu.VMEM((chunks, CW), jnp.int32),
            pltpu.VMEM((CW, d2), jnp.float32),
            pltpu.VMEM((CW, d2), jnp.float32),
            pltpu.VMEM((CW, d2), jnp.float32),
            pltpu.SemaphoreType.DMA,
            pltpu.SemaphoreType.DMA,
            pltpu.VMEM_SHARED((NPAD, d2), jnp.float32),
        ],
    )


def _mm_scale_body(x_ref, w_ref, deg_ref, y_ref, dinv_ref):
    xw = jnp.dot(x_ref[...], w_ref[...], preferred_element_type=jnp.float32)
    deg = deg_ref[0] + deg_ref[1] + 1.0
    dinv = lax.rsqrt(deg)
    y_ref[...] = xw * dinv
    dinv_ref[...] = dinv


def _layer2_body(acc_ref, y1_ref, dinv_ref, b1_ref, w2_ref, y2_ref):
    dinv = dinv_ref[...]
    agg = jnp.concatenate([acc_ref[0], acc_ref[1]], axis=1)
    pre = (agg + y1_ref[...]) * dinv + b1_ref[...]
    h = jnp.maximum(pre, 0.0)
    y2_ref[...] = (
        jnp.dot(h, w2_ref[...], preferred_element_type=jnp.float32) * dinv
    )


def _final_body(acc_ref, y2_ref, dinv_ref, b2_ref, out_ref):
    agg = jnp.concatenate([acc_ref[0], acc_ref[1]], axis=1)
    out_ref[...] = (agg + y2_ref[...]) * dinv_ref[...] + b2_ref[...]


def _mm_scale_call(xp, w1, degp, di, dh):
    return pl.pallas_call(
        _mm_scale_body,
        grid=(NPAD // BM,),
        in_specs=[
            pl.BlockSpec((BM, di), lambda i: (i, 0)),
            pl.BlockSpec((di, dh), lambda i: (0, 0)),
            pl.BlockSpec((NC, BM, 1), lambda i: (0, i, 0)),
        ],
        out_specs=[
            pl.BlockSpec((BM, dh), lambda i: (i, 0)),
            pl.BlockSpec((BM, 1), lambda i: (i, 0)),
        ],
        out_shape=[
            jax.ShapeDtypeStruct((NPAD, dh), jnp.float32),
            jax.ShapeDtypeStruct((NPAD, 1), jnp.float32),
        ],
    )(xp, w1, degp)


def _layer2_call(acc, y1, dinv, b1, w2, dh, do):
    return pl.pallas_call(
        _layer2_body,
        grid=(NPAD // BM,),
        in_specs=[
            pl.BlockSpec((NC, BM, dh // 2), lambda i: (0, i, 0)),
            pl.BlockSpec((BM, dh), lambda i: (i, 0)),
            pl.BlockSpec((BM, 1), lambda i: (i, 0)),
            pl.BlockSpec((1, dh), lambda i: (0, 0)),
            pl.BlockSpec((dh, do), lambda i: (0, 0)),
        ],
        out_specs=pl.BlockSpec((BM, do), lambda i: (i, 0)),
        out_shape=jax.ShapeDtypeStruct((NPAD, do), jnp.float32),
    )(acc, y1, dinv, b1, w2)


def _final_call(acc, y2, dinv, b2, do):
    return pl.pallas_call(
        _final_body,
        grid=(NPAD // BM,),
        in_specs=[
            pl.BlockSpec((NC, BM, do // 2), lambda i: (0, i, 0)),
            pl.BlockSpec((BM, do), lambda i: (i, 0)),
            pl.BlockSpec((BM, 1), lambda i: (i, 0)),
            pl.BlockSpec((1, do), lambda i: (0, 0)),
        ],
        out_specs=pl.BlockSpec((BM, do), lambda i: (i, 0)),
        out_shape=jax.ShapeDtypeStruct((NPAD, do), jnp.float32),
    )(acc, y2, dinv, b2)


def kernel(x, edge_index, W1, b1, W2, b2):
    di = x.shape[1]
    dh = W1.shape[1]
    do = W2.shape[1]
    e = edge_index.shape[1]
    ew = 2 * NS * CW
    ep = -(-e // ew) * ew
    chunks_sc = ep // (NS * CW)
    chunks_deg = chunks_sc // NC

    src = edge_index[0]
    dst = edge_index[1]
    pad = jnp.full((ep - e,), N, jnp.int32)
    srcp = jnp.concatenate([src, pad])
    dstp = jnp.concatenate([dst, pad])
    src_e = (2 * srcp).reshape(NS, chunks_sc, CW)
    src_o = (2 * srcp + 1).reshape(NS, chunks_sc, CW)
    dst_sc = dstp.reshape(NS, chunks_sc, CW)
    dst_deg = dstp.reshape(NW, chunks_deg, CW)
    xp = jnp.zeros((NPAD, di), jnp.float32).at[:N, :].set(x)
    zeros2d = jnp.zeros((CW, dh // 2), jnp.float32)
    zeros1d = jnp.zeros((CW, CW), jnp.float32)
    ones1d = jnp.ones((CW,), jnp.float32)

    deg_fn = _make_deg_kernel(chunks_deg)
    scat_fn = _make_scatter_kernel(chunks_sc, dh)

    degp = deg_fn(dst_deg, zeros1d, ones1d)
    degp3 = degp.reshape(NC, NPAD, 1)
    y1, dinv = _mm_scale_call(xp, W1, degp3, di, dh)
    y1v = y1.reshape(2 * NPAD, dh // 2)
    acc1 = scat_fn(y1v, src_e, src_o, dst_sc, zeros2d)
    y2 = _layer2_call(acc1, y1, dinv, b1.reshape(1, dh), W2, dh, do)
    y2v = y2.reshape(2 * NPAD, do // 2)
    acc2 = scat_fn(y2v, src_e, src_o, dst_sc, zeros2d)
    outp = _final_call(acc2, y2, dinv, b2.reshape(1, do), do)
    return outp[:N]
